# baseline (speedup 1.0000x reference)
import functools

import jax
import jax.numpy as jnp
from jax import lax
from jax.experimental import pallas as pl
from jax.experimental.pallas import tpu as pltpu
from jax.experimental.pallas import tpu_sc as plsc

B = 16384
EMB = 32
HID = 64
NC = 2
NS = 16
NW = NC * NS
CHUNK = 128
NCHUNK = B // CHUNK
CPW = NCHUNK // NW

_mesh = plsc.VectorSubcoreMesh(core_axis_name="c", subcore_axis_name="s")


@functools.partial(
    pl.kernel,
    out_type=jax.ShapeDtypeStruct((2, NCHUNK, CHUNK, EMB), jnp.float32),
    mesh=_mesh,
    compiler_params=pltpu.CompilerParams(use_tc_tiling_on_sc=False),
    scratch_types=[
        pltpu.VMEM((CPW, CHUNK), jnp.int32),
        pltpu.VMEM((CPW, CHUNK), jnp.int32),
        pltpu.VMEM((CPW, CHUNK, EMB), jnp.float32),
        pltpu.VMEM((CPW, CHUNK, EMB), jnp.float32),
        pltpu.SemaphoreType.DMA,
    ],
)
def _sc_gather(idx_hbm, lon_hbm, lat_hbm, out_hbm,
               idx_lon, idx_lat, rows_lon, rows_lat, sem):
    wid = lax.axis_index("s") * NC + lax.axis_index("c")
    base = wid * CPW
    pltpu.sync_copy(idx_hbm.at[0, pl.ds(base, CPW)], idx_lon)
    pltpu.sync_copy(idx_hbm.at[1, pl.ds(base, CPW)], idx_lat)
    copies = []
    for j in range(CPW):
        copies.append(pltpu.async_copy(lon_hbm.at[idx_lon.at[j]], rows_lon.at[j], sem))
        copies.append(pltpu.async_copy(lat_hbm.at[idx_lat.at[j]], rows_lat.at[j], sem))
    for c in copies:
        c.wait()
    pltpu.sync_copy(rows_lon, out_hbm.at[0, pl.ds(base, CPW)])
    pltpu.sync_copy(rows_lat, out_hbm.at[1, pl.ds(base, CPW)])


BM = 2048
GM = BM // CHUNK


def _mm_body(x_ref, wt_ref, b_ref, o_ref):
    x0 = x_ref[0].reshape(BM, EMB)
    x1 = x_ref[1].reshape(BM, EMB)
    acc = jnp.dot(x0, wt_ref[:EMB, :], preferred_element_type=jnp.float32)
    acc = acc + jnp.dot(x1, wt_ref[EMB:, :], preferred_element_type=jnp.float32)
    o_ref[...] = acc + b_ref[...]


def _tc_project(emb, wt, b2):
    return pl.pallas_call(
        _mm_body,
        grid=(B // BM,),
        in_specs=[
            pl.BlockSpec((2, GM, CHUNK, EMB), lambda i: (0, i, 0, 0)),
            pl.BlockSpec((2 * EMB, HID), lambda i: (0, 0)),
            pl.BlockSpec((1, HID), lambda i: (0, 0)),
        ],
        out_specs=pl.BlockSpec((BM, HID), lambda i: (i, 0)),
        out_shape=jax.ShapeDtypeStruct((B, HID), jnp.float32),
    )(emb, wt, b2)


def kernel(batch_seq_cat, lon_table, lat_table, W, b):
    idx_t = batch_seq_cat.T.reshape(2, NCHUNK, CHUNK)
    emb = _sc_gather(idx_t, lon_table, lat_table)
    return _tc_project(emb, W.T, b.reshape(1, HID))
